# Initial kernel scaffold; baseline (speedup 1.0000x reference)
import jax
import jax.numpy as jnp
from jax.experimental import pallas as pl


def kernel(feature, pos, W, b):
    raise NotImplementedError("write your pallas kernel here")


# speedup vs baseline: 7.9046x; 7.9046x over previous
import jax
import jax.numpy as jnp
from jax.experimental import pallas as pl
from jax.experimental.pallas import tpu as pltpu

B, N, C, D, K = 8, 4096, 64, 128, 32
M = N // 4
QT = 256


def _fps_body(pos_ref, out_ref, dist_ref):
    x = pos_ref[0]
    y = pos_ref[1]
    z = pos_ref[2]
    lane = jax.lax.broadcasted_iota(jnp.int32, (B, N), 1)
    lane_m = jax.lax.broadcasted_iota(jnp.int32, (B, M), 1)
    dist_ref[...] = jnp.full((B, N), 1e10, jnp.float32)

    def body(i, first):
        oh = lane == first
        cx = jnp.sum(jnp.where(oh, x, 0.0), axis=1, keepdims=True)
        cy = jnp.sum(jnp.where(oh, y, 0.0), axis=1, keepdims=True)
        cz = jnp.sum(jnp.where(oh, z, 0.0), axis=1, keepdims=True)
        sel = lane_m == i
        out_ref[0] = jnp.where(sel, cx, out_ref[0])
        out_ref[1] = jnp.where(sel, cy, out_ref[1])
        out_ref[2] = jnp.where(sel, cz, out_ref[2])
        dx = x - cx
        dy = y - cy
        dz = z - cz
        d = (dx * dx + dy * dy) + dz * dz
        dist = jnp.minimum(dist_ref[...], d)
        dist_ref[...] = dist
        mx = jnp.max(dist, axis=1, keepdims=True)
        return jnp.min(jnp.where(dist == mx, lane, N), axis=1, keepdims=True)

    jax.lax.fori_loop(0, M, body, jnp.zeros((B, 1), jnp.int32))


def _knn_body(pos_ref, q_ref, f_ref, w_ref, b_ref, out_ref, h_ref, d_ref):
    bi = pl.program_id(0)
    qi = pl.program_id(1)

    @pl.when(qi == 0)
    def _():
        h = jnp.dot(f_ref[0], w_ref[...], preferred_element_type=jnp.float32)
        h_ref[...] = jnp.maximum(h + b_ref[...], 0.0)

    x = pos_ref[0, 0:1]
    y = pos_ref[0, 1:2]
    z = pos_ref[0, 2:3]
    lane8 = jax.lax.broadcasted_iota(jnp.int32, (QT, B), 1)
    bsel = lane8 == bi
    qx = jnp.sum(jnp.where(bsel, q_ref[0], 0.0), axis=1, keepdims=True)
    qy = jnp.sum(jnp.where(bsel, q_ref[1], 0.0), axis=1, keepdims=True)
    qz = jnp.sum(jnp.where(bsel, q_ref[2], 0.0), axis=1, keepdims=True)
    dx = qx - x
    dy = qy - y
    dz = qz - z
    d_ref[...] = (dx * dx + dy * dy) + dz * dz
    lane = jax.lax.broadcasted_iota(jnp.int32, (QT, N), 1)

    def step(j, acc):
        d = d_ref[...]
        m = jnp.min(d, axis=1, keepdims=True)
        first = jnp.min(jnp.where(d == m, lane, N), axis=1, keepdims=True)
        oh = lane == first
        row = jnp.dot(oh.astype(jnp.float32), h_ref[...],
                      preferred_element_type=jnp.float32)
        d_ref[...] = jnp.where(oh, jnp.inf, d)
        return jnp.maximum(acc, row)

    out_ref[0] = jax.lax.fori_loop(
        0, K, step, jnp.zeros((QT, D), jnp.float32))


def kernel(feature, pos, W, b):
    pos_t = jnp.transpose(pos, (2, 0, 1))

    sampled_c = pl.pallas_call(
        _fps_body,
        out_shape=jax.ShapeDtypeStruct((3, B, M), jnp.float32),
        scratch_shapes=[pltpu.VMEM((B, N), jnp.float32)],
    )(pos_t)

    sampled_pos = jnp.transpose(sampled_c, (1, 2, 0))
    q_cols = jnp.transpose(sampled_c, (0, 2, 1))

    output = pl.pallas_call(
        _knn_body,
        grid=(B, M // QT),
        in_specs=[
            pl.BlockSpec((1, 3, N), lambda bi, qi: (bi, 0, 0)),
            pl.BlockSpec((3, QT, B), lambda bi, qi: (0, qi, 0)),
            pl.BlockSpec((1, N, C), lambda bi, qi: (bi, 0, 0)),
            pl.BlockSpec((C, D), lambda bi, qi: (0, 0)),
            pl.BlockSpec((1, D), lambda bi, qi: (0, 0)),
        ],
        out_specs=pl.BlockSpec((1, QT, D), lambda bi, qi: (bi, qi, 0)),
        out_shape=jax.ShapeDtypeStruct((B, M, D), jnp.float32),
        scratch_shapes=[
            pltpu.VMEM((N, D), jnp.float32),
            pltpu.VMEM((QT, N), jnp.float32),
        ],
    )(jnp.transpose(pos, (0, 2, 1)), q_cols, feature, W, b.reshape(1, D))

    return (output, sampled_pos)
